# baseline (speedup 1.0000x reference)
import functools

import jax
import jax.numpy as jnp
from jax import lax
from jax.experimental import pallas as pl
from jax.experimental.pallas import tpu as pltpu
from jax.experimental.pallas import tpu_sc as plsc

N = 10000
E = 320000
D = 128
H = 16
G = 64
NPAD = 10240
CHUNK = 128
NCHUNKS = E // CHUNK
NW = 32
K78 = NCHUNKS // NW
TAIL = NCHUNKS - K78 * NW
NBUF = 4
RPS = NPAD // 16
BLK = 256
GRID = NPAD // BLK

_mesh = plsc.VectorSubcoreMesh(core_axis_name="c", subcore_axis_name="s")
_sc_params = pltpu.CompilerParams(use_tc_tiling_on_sc=False,
                                  needs_layout_passes=False)


@functools.partial(
    pl.kernel,
    out_type=jax.ShapeDtypeStruct((2, NPAD * H), jnp.float32),
    mesh=_mesh,
    compiler_params=_sc_params,
    scratch_types=[
        pltpu.VMEM_SHARED((NPAD,), jnp.float32),
        pltpu.VMEM((K78 + 1, CHUNK), jnp.int32),
        pltpu.VMEM((CHUNK,), jnp.float32),
        pltpu.VMEM((640,), jnp.float32),
        pltpu.VMEM((640 * H,), jnp.float32),
        pltpu.SemaphoreType.DMA,
        pltpu.SemaphoreType.DMA,
        pltpu.SemaphoreType.DMA,
        pltpu.SemaphoreType.DMA,
    ],
)
def _deg(dst2_hbm, zeros1_hbm, degx_hbm, deg_sh, didx, ones_v, dval_v, dexp_v,
         sm0, sm1, sm2, sm3):
    ssem = (sm0, sm1, sm2, sm3)
    c = lax.axis_index("c")
    s = lax.axis_index("s")
    wid = s * 2 + c
    for k in range(CHUNK // 16):
        ones_v[pl.ds(k * 16, 16)] = jnp.full((16,), 1.0, jnp.float32)
    pltpu.sync_copy(zeros1_hbm.at[pl.ds(s * 640, 640)],
                    deg_sh.at[pl.ds(s * 640, 640)])
    pltpu.sync_copy(dst2_hbm.at[pl.ds(wid * K78, K78)],
                    didx.at[pl.ds(0, K78)])

    @pl.when(wid < TAIL)
    def _():
        pltpu.sync_copy(dst2_hbm.at[K78 * NW + wid], didx.at[K78])

    kmax = jnp.where(wid < TAIL, K78 + 1, K78)
    plsc.subcore_barrier()

    def grp_body(gidx, carry):
        for b in range(NBUF):
            j = gidx * NBUF + b

            @pl.when(j < kmax)
            def _(b=b, j=j):
                @pl.when(j >= NBUF)
                def _():
                    pltpu.make_async_copy(
                        ones_v, deg_sh.at[didx.at[0]], ssem[b]).wait()
                pltpu.async_copy(ones_v, deg_sh.at[didx.at[j]], ssem[b],
                                 add=True)
        return carry

    lax.fori_loop(0, (K78 + 1 + NBUF - 1) // NBUF, grp_body, 0)
    for b in range(NBUF):
        pltpu.make_async_copy(ones_v, deg_sh.at[didx.at[0]], ssem[b]).wait()
    plsc.subcore_barrier()

    pltpu.sync_copy(deg_sh.at[pl.ds(s * 640, 640)], dval_v)
    lane_iota = lax.iota(jnp.int32, 16)

    def exp_body(ci, carry):
        idx = lane_iota * 0 + ci
        val = plsc.load_gather(dval_v, [idx])
        dexp_v[pl.ds(ci * H, H)] = val
        return carry

    lax.fori_loop(0, 640, exp_body, 0)

    pltpu.sync_copy(dexp_v,
                    degx_hbm.at[c].at[pl.ds(s * 640 * H, 640 * H)])


@functools.partial(
    pl.kernel,
    out_type=jax.ShapeDtypeStruct((2, NPAD, H), jnp.float32),
    mesh=_mesh,
    compiler_params=_sc_params,
    scratch_types=[
        pltpu.VMEM_SHARED((NPAD, H), jnp.float32),
        pltpu.VMEM((K78 + 1, CHUNK), jnp.int32),
        pltpu.VMEM((K78 + 1, CHUNK), jnp.int32),
        pltpu.VMEM((CHUNK, H), jnp.float32),
        pltpu.VMEM((CHUNK, H), jnp.float32),
        pltpu.VMEM((CHUNK, H), jnp.float32),
        pltpu.VMEM((CHUNK, H), jnp.float32),
        pltpu.SemaphoreType.DMA,
        pltpu.SemaphoreType.DMA,
        pltpu.SemaphoreType.DMA,
        pltpu.SemaphoreType.DMA,
        pltpu.SemaphoreType.DMA,
        pltpu.SemaphoreType.DMA,
        pltpu.SemaphoreType.DMA,
        pltpu.SemaphoreType.DMA,
    ],
)
def _msg(src2_hbm, dst2_hbm, g_hbm, zeros2_hbm, sp_hbm,
         acc_sh, sidx, didx, r0, r1, r2, r3,
         gm0, gm1, gm2, gm3, sm0, sm1, sm2, sm3):
    rows = (r0, r1, r2, r3)
    gsem = (gm0, gm1, gm2, gm3)
    ssem = (sm0, sm1, sm2, sm3)
    c = lax.axis_index("c")
    s = lax.axis_index("s")
    wid = s * 2 + c
    pltpu.sync_copy(zeros2_hbm.at[pl.ds(s * RPS, RPS)],
                    acc_sh.at[pl.ds(s * RPS, RPS)])
    pltpu.sync_copy(src2_hbm.at[pl.ds(wid * K78, K78)],
                    sidx.at[pl.ds(0, K78)])
    pltpu.sync_copy(dst2_hbm.at[pl.ds(wid * K78, K78)],
                    didx.at[pl.ds(0, K78)])

    @pl.when(wid < TAIL)
    def _():
        pltpu.sync_copy(src2_hbm.at[K78 * NW + wid], sidx.at[K78])
        pltpu.sync_copy(dst2_hbm.at[K78 * NW + wid], didx.at[K78])

    kmax = jnp.where(wid < TAIL, K78 + 1, K78)
    plsc.subcore_barrier()

    def issue_gather(j, b):
        pltpu.async_copy(g_hbm.at[sidx.at[j]], rows[b], gsem[b])

    def wait_gather(b):
        pltpu.make_async_copy(g_hbm.at[sidx.at[0]], rows[b], gsem[b]).wait()

    def issue_scatter(j, b):
        pltpu.async_copy(rows[b], acc_sh.at[didx.at[j]], ssem[b], add=True)

    def wait_scatter(b):
        pltpu.make_async_copy(rows[b], acc_sh.at[didx.at[0]], ssem[b]).wait()

    for b in range(NBUF):
        issue_gather(jnp.int32(b), b)

    def grp_body(gidx, carry):
        for b in range(NBUF):
            j = gidx * NBUF + b

            @pl.when(j < kmax)
            def _(b=b, j=j):
                wait_gather(b)
                issue_scatter(j, b)
                jn = j + NBUF

                @pl.when(jn < kmax)
                def _(b=b, jn=jn):
                    wait_scatter(b)
                    issue_gather(jn, b)
        return carry

    lax.fori_loop(0, (K78 + 1 + NBUF - 1) // NBUF, grp_body, 0)
    for b in range(NBUF):
        wait_scatter(b)
    plsc.subcore_barrier()
    pltpu.sync_copy(acc_sh.at[pl.ds(s * RPS, RPS)],
                    sp_hbm.at[c].at[pl.ds(s * RPS, RPS)])


def _mm1_body(x_ref, w1_ref, d0_ref, d1_ref, g_ref, dinv_ref):
    deg = d0_ref[...] + d1_ref[...] + 1.0
    dinv = lax.rsqrt(jnp.maximum(deg, 1.0))
    h = jnp.dot(x_ref[...], w1_ref[...], preferred_element_type=jnp.float32)
    g_ref[...] = h * dinv
    dinv_ref[...] = dinv


_mm1 = pl.pallas_call(
    _mm1_body,
    grid=(GRID,),
    in_specs=[
        pl.BlockSpec((BLK, D), lambda i: (i, 0)),
        pl.BlockSpec((D, H), lambda i: (0, 0)),
        pl.BlockSpec((BLK, H), lambda i: (i, 0)),
        pl.BlockSpec((BLK, H), lambda i: (i, 0)),
    ],
    out_specs=[
        pl.BlockSpec((BLK, H), lambda i: (i, 0)),
        pl.BlockSpec((BLK, H), lambda i: (i, 0)),
    ],
    out_shape=[
        jax.ShapeDtypeStruct((NPAD, H), jnp.float32),
        jax.ShapeDtypeStruct((NPAD, H), jnp.float32),
    ],
)


def _comb1_body(s0_ref, s1_ref, g_ref, dinv_ref, w2_ref, b1_ref, g2_ref):
    h1 = jnp.maximum(
        (s0_ref[...] + s1_ref[...] + g_ref[...]) * dinv_ref[...] + b1_ref[...],
        0.0)
    g2_ref[...] = jnp.dot(h1, w2_ref[...],
                          preferred_element_type=jnp.float32) * dinv_ref[...]


_comb1 = pl.pallas_call(
    _comb1_body,
    grid=(GRID,),
    in_specs=[
        pl.BlockSpec((BLK, H), lambda i: (i, 0)),
        pl.BlockSpec((BLK, H), lambda i: (i, 0)),
        pl.BlockSpec((BLK, H), lambda i: (i, 0)),
        pl.BlockSpec((BLK, H), lambda i: (i, 0)),
        pl.BlockSpec((H, H), lambda i: (0, 0)),
        pl.BlockSpec((1, H), lambda i: (0, 0)),
    ],
    out_specs=pl.BlockSpec((BLK, H), lambda i: (i, 0)),
    out_shape=jax.ShapeDtypeStruct((NPAD, H), jnp.float32),
)


def _comb2_body(q0_ref, q1_ref, g2_ref, dinv_ref, b2_ref, batch_ref,
                wl_ref, bl_ref, out_ref, pool_acc, cnt_acc):
    i = pl.program_id(0)

    @pl.when(i == 0)
    def _():
        pool_acc[...] = jnp.zeros_like(pool_acc)
        cnt_acc[...] = jnp.zeros_like(cnt_acc)

    h2 = jnp.maximum(
        (q0_ref[...] + q1_ref[...] + g2_ref[...]) * dinv_ref[...] + b2_ref[...],
        0.0)
    iota = lax.broadcasted_iota(jnp.int32, (BLK, G), 1)
    onehot = (batch_ref[:, :1] == iota).astype(jnp.float32)
    pool_acc[...] += lax.dot_general(
        onehot, h2, (((0,), (0,)), ((), ())),
        preferred_element_type=jnp.float32)
    cnt_acc[...] += lax.dot_general(
        onehot, jnp.ones((BLK, 1), jnp.float32), (((0,), (0,)), ((), ())),
        preferred_element_type=jnp.float32)

    @pl.when(i == GRID - 1)
    def _():
        pooled = pool_acc[...] / jnp.maximum(cnt_acc[...], 1.0)
        z = jnp.dot(pooled, wl_ref[...],
                    preferred_element_type=jnp.float32) + bl_ref[...]
        out_ref[...] = jax.nn.sigmoid(z)


_comb2 = pl.pallas_call(
    _comb2_body,
    grid=(GRID,),
    in_specs=[
        pl.BlockSpec((BLK, H), lambda i: (i, 0)),
        pl.BlockSpec((BLK, H), lambda i: (i, 0)),
        pl.BlockSpec((BLK, H), lambda i: (i, 0)),
        pl.BlockSpec((BLK, H), lambda i: (i, 0)),
        pl.BlockSpec((1, H), lambda i: (0, 0)),
        pl.BlockSpec((BLK, H), lambda i: (i, 0)),
        pl.BlockSpec((H, 1), lambda i: (0, 0)),
        pl.BlockSpec((1, 1), lambda i: (0, 0)),
    ],
    out_specs=pl.BlockSpec((G, 1), lambda i: (0, 0)),
    out_shape=jax.ShapeDtypeStruct((G, 1), jnp.float32),
    scratch_shapes=[
        pltpu.VMEM((G, H), jnp.float32),
        pltpu.VMEM((G, 1), jnp.float32),
    ],
)


def kernel(x, edge_index, batch, W1, b1, W2, b2, Wl, bl):
    src2 = edge_index[0].reshape(NCHUNKS, CHUNK)
    dst2 = edge_index[1].reshape(NCHUNKS, CHUNK)
    xpad = jnp.pad(x, ((0, NPAD - N), (0, 0)))
    batch16 = jnp.broadcast_to(
        jnp.pad(batch, (0, NPAD - N), constant_values=G + 1).reshape(NPAD, 1),
        (NPAD, H))
    zeros1 = jnp.zeros((NPAD,), jnp.float32)
    zeros2 = jnp.zeros((NPAD, H), jnp.float32)

    degx = _deg(dst2, zeros1)
    g1, dinv = _mm1(xpad, W1, degx[0].reshape(NPAD, H),
                    degx[1].reshape(NPAD, H))

    sp1 = _msg(src2, dst2, g1, zeros2)
    g2 = _comb1(sp1[0], sp1[1], g1, dinv, W2, b1.reshape(1, H))

    sp2 = _msg(src2, dst2, g2, zeros2)
    out2d = _comb2(sp2[0], sp2[1], g2, dinv, b2.reshape(1, H), batch16,
                   Wl, bl.reshape(1, 1))
    return out2d[:, 0]


# speedup vs baseline: 1.5227x; 1.5227x over previous
import functools

import jax
import jax.numpy as jnp
from jax import lax
from jax.experimental import pallas as pl
from jax.experimental.pallas import tpu as pltpu
from jax.experimental.pallas import tpu_sc as plsc

N = 10000
E = 320000
D = 128
H = 16
G = 64
NPAD = 10240
CHUNK = 128
NCHUNKS = E // CHUNK
NW = 32
K78 = NCHUNKS // NW
TAIL = NCHUNKS - K78 * NW
NBUF = 4
RPS = NPAD // 16

_mesh = plsc.VectorSubcoreMesh(core_axis_name="c", subcore_axis_name="s")
_sc_params = pltpu.CompilerParams(use_tc_tiling_on_sc=False,
                                  needs_layout_passes=False)

_f32 = jnp.float32


@functools.partial(
    pl.kernel,
    out_type=[jax.ShapeDtypeStruct((NPAD, H), _f32),
              jax.ShapeDtypeStruct((NPAD, H), _f32)],
    mesh=_mesh,
    compiler_params=_sc_params,
    scratch_types=[
        pltpu.VMEM_SHARED((NPAD,), _f32),
        pltpu.VMEM((K78 + 1, CHUNK), jnp.int32),
        pltpu.VMEM((CHUNK,), _f32),
        pltpu.VMEM((640,), _f32),
        pltpu.VMEM((640, H), _f32),
        pltpu.SemaphoreType.DMA,
        pltpu.SemaphoreType.DMA,
        pltpu.SemaphoreType.DMA,
        pltpu.SemaphoreType.DMA,
    ],
)
def _deg(ei3_hbm, zeros1_hbm, degx0_hbm, degx1_hbm, deg_sh, didx, ones_v,
         dval_v, dexp_v, sm0, sm1, sm2, sm3):
    ssem = (sm0, sm1, sm2, sm3)
    c = lax.axis_index("c")
    s = lax.axis_index("s")
    wid = s * 2 + c
    for k in range(CHUNK // 16):
        ones_v[pl.ds(k * 16, 16)] = jnp.full((16,), 1.0, _f32)
    pltpu.sync_copy(zeros1_hbm.at[pl.ds(s * 640, 640)],
                    deg_sh.at[pl.ds(s * 640, 640)])
    pltpu.sync_copy(ei3_hbm.at[1].at[pl.ds(wid * K78, K78)],
                    didx.at[pl.ds(0, K78)])

    @pl.when(wid < TAIL)
    def _():
        pltpu.sync_copy(ei3_hbm.at[1].at[K78 * NW + wid], didx.at[K78])

    kmax = jnp.where(wid < TAIL, K78 + 1, K78)
    plsc.subcore_barrier()

    def grp_body(gidx, carry):
        for b in range(NBUF):
            j = gidx * NBUF + b

            @pl.when(j < kmax)
            def _(b=b, j=j):
                @pl.when(j >= NBUF)
                def _():
                    pltpu.make_async_copy(
                        ones_v, deg_sh.at[didx.at[0]], ssem[b]).wait()
                pltpu.async_copy(ones_v, deg_sh.at[didx.at[j]], ssem[b],
                                 add=True)
        return carry

    lax.fori_loop(0, (K78 + 1 + NBUF - 1) // NBUF, grp_body, 0)
    for b in range(NBUF):
        pltpu.make_async_copy(ones_v, deg_sh.at[didx.at[0]], ssem[b]).wait()
    plsc.subcore_barrier()

    pltpu.sync_copy(deg_sh.at[pl.ds(s * 640, 640)], dval_v)
    lane_iota = lax.iota(jnp.int32, 16)

    def exp_body(ci, carry):
        idx = lane_iota * 0 + ci
        val = plsc.load_gather(dval_v, [idx])
        dexp_v[ci, :] = val
        return carry

    lax.fori_loop(0, 640, exp_body, 0)

    @pl.when(c == 0)
    def _():
        pltpu.sync_copy(dexp_v, degx0_hbm.at[pl.ds(s * 640, 640)])

    @pl.when(c == 1)
    def _():
        pltpu.sync_copy(dexp_v, degx1_hbm.at[pl.ds(s * 640, 640)])


@functools.partial(
    pl.kernel,
    out_type=[jax.ShapeDtypeStruct((NPAD, H), _f32),
              jax.ShapeDtypeStruct((NPAD, H), _f32)],
    mesh=_mesh,
    compiler_params=_sc_params,
    scratch_types=[
        pltpu.VMEM_SHARED((NPAD, H), _f32),
        pltpu.VMEM((K78 + 1, CHUNK), jnp.int32),
        pltpu.VMEM((K78 + 1, CHUNK), jnp.int32),
        pltpu.VMEM((CHUNK, H), _f32),
        pltpu.VMEM((CHUNK, H), _f32),
        pltpu.VMEM((CHUNK, H), _f32),
        pltpu.VMEM((CHUNK, H), _f32),
        pltpu.SemaphoreType.DMA,
        pltpu.SemaphoreType.DMA,
        pltpu.SemaphoreType.DMA,
        pltpu.SemaphoreType.DMA,
        pltpu.SemaphoreType.DMA,
        pltpu.SemaphoreType.DMA,
        pltpu.SemaphoreType.DMA,
        pltpu.SemaphoreType.DMA,
    ],
)
def _msg(ei3_hbm, g_hbm, zeros2_hbm, sp0_hbm, sp1_hbm,
         acc_sh, sidx, didx, r0, r1, r2, r3,
         gm0, gm1, gm2, gm3, sm0, sm1, sm2, sm3):
    rows = (r0, r1, r2, r3)
    gsem = (gm0, gm1, gm2, gm3)
    ssem = (sm0, sm1, sm2, sm3)
    c = lax.axis_index("c")
    s = lax.axis_index("s")
    wid = s * 2 + c
    pltpu.sync_copy(zeros2_hbm.at[pl.ds(s * RPS, RPS)],
                    acc_sh.at[pl.ds(s * RPS, RPS)])
    pltpu.sync_copy(ei3_hbm.at[0].at[pl.ds(wid * K78, K78)],
                    sidx.at[pl.ds(0, K78)])
    pltpu.sync_copy(ei3_hbm.at[1].at[pl.ds(wid * K78, K78)],
                    didx.at[pl.ds(0, K78)])

    @pl.when(wid < TAIL)
    def _():
        pltpu.sync_copy(ei3_hbm.at[0].at[K78 * NW + wid], sidx.at[K78])
        pltpu.sync_copy(ei3_hbm.at[1].at[K78 * NW + wid], didx.at[K78])

    kmax = jnp.where(wid < TAIL, K78 + 1, K78)
    plsc.subcore_barrier()

    def issue_gather(j, b):
        pltpu.async_copy(g_hbm.at[sidx.at[j]], rows[b], gsem[b])

    def wait_gather(b):
        pltpu.make_async_copy(g_hbm.at[sidx.at[0]], rows[b], gsem[b]).wait()

    def issue_scatter(j, b):
        pltpu.async_copy(rows[b], acc_sh.at[didx.at[j]], ssem[b], add=True)

    def wait_scatter(b):
        pltpu.make_async_copy(rows[b], acc_sh.at[didx.at[0]], ssem[b]).wait()

    for b in range(NBUF):
        issue_gather(jnp.int32(b), b)

    def grp_body(gidx, carry):
        for b in range(NBUF):
            j = gidx * NBUF + b

            @pl.when(j < kmax)
            def _(b=b, j=j):
                wait_gather(b)
                issue_scatter(j, b)
                jn = j + NBUF

                @pl.when(jn < kmax)
                def _(b=b, jn=jn):
                    wait_scatter(b)
                    issue_gather(jn, b)
        return carry

    lax.fori_loop(0, (K78 + 1 + NBUF - 1) // NBUF, grp_body, 0)
    for b in range(NBUF):
        wait_scatter(b)
    plsc.subcore_barrier()

    @pl.when(c == 0)
    def _():
        pltpu.sync_copy(acc_sh.at[pl.ds(s * RPS, RPS)],
                        sp0_hbm.at[pl.ds(s * RPS, RPS)])

    @pl.when(c == 1)
    def _():
        pltpu.sync_copy(acc_sh.at[pl.ds(s * RPS, RPS)],
                        sp1_hbm.at[pl.ds(s * RPS, RPS)])


def _mm1_body(x_ref, w1_ref, d0_ref, d1_ref, g_ref, dinv_ref):
    deg = d0_ref[...] + d1_ref[...] + 1.0
    dinv = lax.rsqrt(jnp.maximum(deg, 1.0))
    h = jnp.dot(x_ref[...], w1_ref[...], preferred_element_type=_f32)
    g_ref[...] = h * dinv
    dinv_ref[...] = dinv


_mm1 = pl.pallas_call(
    _mm1_body,
    out_shape=[
        jax.ShapeDtypeStruct((NPAD, H), _f32),
        jax.ShapeDtypeStruct((NPAD, H), _f32),
    ],
)


def _comb1_body(s0_ref, s1_ref, g_ref, dinv_ref, w2_ref, b1_ref, g2_ref):
    h1 = jnp.maximum(
        (s0_ref[...] + s1_ref[...] + g_ref[...]) * dinv_ref[...] + b1_ref[...],
        0.0)
    g2_ref[...] = jnp.dot(h1, w2_ref[...],
                          preferred_element_type=_f32) * dinv_ref[...]


_comb1 = pl.pallas_call(
    _comb1_body,
    out_shape=jax.ShapeDtypeStruct((NPAD, H), _f32),
)


def _comb2_body(q0_ref, q1_ref, g2_ref, dinv_ref, b2_ref, batch_ref,
                wl_ref, bl_ref, out_ref):
    h2 = jnp.maximum(
        (q0_ref[...] + q1_ref[...] + g2_ref[...]) * dinv_ref[...] + b2_ref[...],
        0.0)
    iota = lax.broadcasted_iota(jnp.int32, (NPAD, G), 1)
    onehot = (batch_ref[:, :1] == iota).astype(_f32)
    pooled = lax.dot_general(onehot, h2, (((0,), (0,)), ((), ())),
                             preferred_element_type=_f32)
    counts = lax.dot_general(onehot, jnp.ones((NPAD, 1), _f32),
                             (((0,), (0,)), ((), ())),
                             preferred_element_type=_f32)
    pooled = pooled / jnp.maximum(counts, 1.0)
    z = jnp.dot(pooled, wl_ref[...], preferred_element_type=_f32) + bl_ref[...]
    out_ref[...] = jax.nn.sigmoid(z)


_comb2 = pl.pallas_call(
    _comb2_body,
    out_shape=jax.ShapeDtypeStruct((G, 1), _f32),
)


def kernel(x, edge_index, batch, W1, b1, W2, b2, Wl, bl):
    ei3 = edge_index.reshape(2, NCHUNKS, CHUNK)
    xpad = jnp.pad(x, ((0, NPAD - N), (0, 0)))
    batch16 = jnp.broadcast_to(
        jnp.pad(batch, (0, NPAD - N), constant_values=G + 1).reshape(NPAD, 1),
        (NPAD, H))
    zeros1 = jnp.zeros((NPAD,), _f32)
    zeros2 = jnp.zeros((NPAD, H), _f32)

    d0, d1 = _deg(ei3, zeros1)
    g1, dinv = _mm1(xpad, W1, d0, d1)

    s0, s1 = _msg(ei3, g1, zeros2)
    g2 = _comb1(s0, s1, g1, dinv, W2, b1.reshape(1, H))

    q0, q1 = _msg(ei3, g2, zeros2)
    out2d = _comb2(q0, q1, g2, dinv, b2.reshape(1, H), batch16,
                   Wl, bl.reshape(1, 1))
    return out2d[:, 0]


# speedup vs baseline: 1.7180x; 1.1282x over previous
import functools

import jax
import jax.numpy as jnp
from jax import lax
from jax.experimental import pallas as pl
from jax.experimental.pallas import tpu as pltpu
from jax.experimental.pallas import tpu_sc as plsc

N = 10000
E = 320000
D = 128
H = 16
G = 64
NPAD = 10240
CHUNK = 128
NCHUNKS = E // CHUNK
NW = 32
K78 = NCHUNKS // NW
TAIL = NCHUNKS - K78 * NW
NBUF = 4
RPS = NPAD // 16

_mesh = plsc.VectorSubcoreMesh(core_axis_name="c", subcore_axis_name="s")
_sc_params = pltpu.CompilerParams(use_tc_tiling_on_sc=False,
                                  needs_layout_passes=False)

_f32 = jnp.float32


@functools.partial(
    pl.kernel,
    out_type=[jax.ShapeDtypeStruct((16, NPAD), _f32),
              jax.ShapeDtypeStruct((16, NPAD), _f32)],
    mesh=_mesh,
    compiler_params=_sc_params,
    scratch_types=[
        pltpu.VMEM_SHARED((NPAD,), _f32),
        pltpu.VMEM((K78 + 1, CHUNK), jnp.int32),
        pltpu.VMEM((CHUNK,), _f32),
        pltpu.VMEM((RPS,), _f32),
        pltpu.VMEM((16, RPS), _f32),
        pltpu.SemaphoreType.DMA,
        pltpu.SemaphoreType.DMA,
        pltpu.SemaphoreType.DMA,
        pltpu.SemaphoreType.DMA,
    ],
)
def _deg(ei3_hbm, zeros1_hbm, degt0_hbm, degt1_hbm, deg_sh, didx, ones_v,
         dval_v, drep_v, sm0, sm1, sm2, sm3):
    ssem = (sm0, sm1, sm2, sm3)
    c = lax.axis_index("c")
    s = lax.axis_index("s")
    wid = s * 2 + c
    for k in range(CHUNK // 16):
        ones_v[pl.ds(k * 16, 16)] = jnp.full((16,), 1.0, _f32)
    pltpu.sync_copy(zeros1_hbm.at[pl.ds(s * RPS, RPS)],
                    deg_sh.at[pl.ds(s * RPS, RPS)])
    pltpu.sync_copy(ei3_hbm.at[1].at[pl.ds(wid * K78, K78)],
                    didx.at[pl.ds(0, K78)])

    @pl.when(wid < TAIL)
    def _():
        pltpu.sync_copy(ei3_hbm.at[1].at[K78 * NW + wid], didx.at[K78])

    kmax = jnp.where(wid < TAIL, K78 + 1, K78)
    plsc.subcore_barrier()

    def grp_body(gidx, carry):
        for b in range(NBUF):
            j = gidx * NBUF + b

            @pl.when(j < kmax)
            def _(b=b, j=j):
                @pl.when(j >= NBUF)
                def _():
                    pltpu.make_async_copy(
                        ones_v, deg_sh.at[didx.at[0]], ssem[b]).wait()
                pltpu.async_copy(ones_v, deg_sh.at[didx.at[j]], ssem[b],
                                 add=True)
        return carry

    lax.fori_loop(0, (K78 + 1 + NBUF - 1) // NBUF, grp_body, 0)
    for b in range(NBUF):
        pltpu.make_async_copy(ones_v, deg_sh.at[didx.at[0]], ssem[b]).wait()
    plsc.subcore_barrier()

    pltpu.sync_copy(deg_sh.at[pl.ds(s * RPS, RPS)], dval_v)

    def rep_body(k, carry):
        v = dval_v[pl.ds(k * 16, 16)]
        for l in range(16):
            drep_v[l, pl.ds(k * 16, 16)] = v
        return carry

    lax.fori_loop(0, RPS // 16, rep_body, 0)

    @pl.when(c == 0)
    def _():
        pltpu.sync_copy(drep_v,
                        degt0_hbm.at[pl.ds(0, 16), pl.ds(s * RPS, RPS)])

    @pl.when(c == 1)
    def _():
        pltpu.sync_copy(drep_v,
                        degt1_hbm.at[pl.ds(0, 16), pl.ds(s * RPS, RPS)])


@functools.partial(
    pl.kernel,
    out_type=[jax.ShapeDtypeStruct((16, NPAD), _f32),
              jax.ShapeDtypeStruct((16, NPAD), _f32),
              jax.ShapeDtypeStruct((NPAD, H), _f32)],
    mesh=_mesh,
    compiler_params=_sc_params,
    scratch_types=[
        pltpu.VMEM_SHARED((NPAD, H), _f32),
        pltpu.VMEM((K78 + 1, CHUNK), jnp.int32),
        pltpu.VMEM((K78 + 1, CHUNK), jnp.int32),
        pltpu.VMEM((16, RPS), _f32),
        pltpu.VMEM((RPS, H), _f32),
        pltpu.VMEM((CHUNK, H), _f32),
        pltpu.VMEM((CHUNK, H), _f32),
        pltpu.VMEM((CHUNK, H), _f32),
        pltpu.VMEM((CHUNK, H), _f32),
        pltpu.SemaphoreType.DMA,
        pltpu.SemaphoreType.DMA,
        pltpu.SemaphoreType.DMA,
        pltpu.SemaphoreType.DMA,
        pltpu.SemaphoreType.DMA,
        pltpu.SemaphoreType.DMA,
        pltpu.SemaphoreType.DMA,
        pltpu.SemaphoreType.DMA,
    ],
)
def _msg(ei3_hbm, gt_hbm, zeros2_hbm, spt0_hbm, spt1_hbm, grows_hbm,
         acc_sh, sidx, didx, tbuf, rbuf, r0, r1, r2, r3,
         gm0, gm1, gm2, gm3, sm0, sm1, sm2, sm3):
    rows = (r0, r1, r2, r3)
    gsem = (gm0, gm1, gm2, gm3)
    ssem = (sm0, sm1, sm2, sm3)
    c = lax.axis_index("c")
    s = lax.axis_index("s")
    wid = s * 2 + c
    lane_iota = lax.iota(jnp.int32, 16)

    pltpu.sync_copy(gt_hbm.at[pl.ds(0, 16), pl.ds(s * RPS, RPS)], tbuf)

    def tr_body(j, carry):
        col = lane_iota * 0 + j
        v = plsc.load_gather(tbuf, [lane_iota, col])
        rbuf[j, :] = v
        return carry

    lax.fori_loop(0, RPS, tr_body, 0)
    pltpu.sync_copy(rbuf, grows_hbm.at[pl.ds(s * RPS, RPS)])

    pltpu.sync_copy(zeros2_hbm.at[pl.ds(s * RPS, RPS)],
                    acc_sh.at[pl.ds(s * RPS, RPS)])
    pltpu.sync_copy(ei3_hbm.at[0].at[pl.ds(wid * K78, K78)],
                    sidx.at[pl.ds(0, K78)])
    pltpu.sync_copy(ei3_hbm.at[1].at[pl.ds(wid * K78, K78)],
                    didx.at[pl.ds(0, K78)])

    @pl.when(wid < TAIL)
    def _():
        pltpu.sync_copy(ei3_hbm.at[0].at[K78 * NW + wid], sidx.at[K78])
        pltpu.sync_copy(ei3_hbm.at[1].at[K78 * NW + wid], didx.at[K78])

    kmax = jnp.where(wid < TAIL, K78 + 1, K78)
    plsc.subcore_barrier()

    def issue_gather(j, b):
        pltpu.async_copy(grows_hbm.at[sidx.at[j]], rows[b], gsem[b])

    def wait_gather(b):
        pltpu.make_async_copy(grows_hbm.at[sidx.at[0]], rows[b],
                              gsem[b]).wait()

    def issue_scatter(j, b):
        pltpu.async_copy(rows[b], acc_sh.at[didx.at[j]], ssem[b], add=True)

    def wait_scatter(b):
        pltpu.make_async_copy(rows[b], acc_sh.at[didx.at[0]], ssem[b]).wait()

    for b in range(NBUF):
        issue_gather(jnp.int32(b), b)

    def grp_body(gidx, carry):
        for b in range(NBUF):
            j = gidx * NBUF + b

            @pl.when(j < kmax)
            def _(b=b, j=j):
                wait_gather(b)
                issue_scatter(j, b)
                jn = j + NBUF

                @pl.when(jn < kmax)
                def _(b=b, jn=jn):
                    wait_scatter(b)
                    issue_gather(jn, b)
        return carry

    lax.fori_loop(0, (K78 + 1 + NBUF - 1) // NBUF, grp_body, 0)
    for b in range(NBUF):
        wait_scatter(b)
    plsc.subcore_barrier()

    pltpu.sync_copy(acc_sh.at[pl.ds(s * RPS, RPS)], rbuf)

    def trb_body(k, carry):
        ridx = k * 16 + lane_iota
        for l in range(16):
            cidx = lane_iota * 0 + l
            v = plsc.load_gather(rbuf, [ridx, cidx])
            tbuf[l, pl.ds(k * 16, 16)] = v
        return carry

    lax.fori_loop(0, RPS // 16, trb_body, 0)

    @pl.when(c == 0)
    def _():
        pltpu.sync_copy(tbuf,
                        spt0_hbm.at[pl.ds(0, 16), pl.ds(s * RPS, RPS)])

    @pl.when(c == 1)
    def _():
        pltpu.sync_copy(tbuf,
                        spt1_hbm.at[pl.ds(0, 16), pl.ds(s * RPS, RPS)])


def _mm1_body(x_ref, w1_ref, d0_ref, d1_ref, gt_ref, dinv_ref):
    deg = d0_ref[...] + d1_ref[...] + 1.0
    dinv = lax.rsqrt(jnp.maximum(deg, 1.0))
    ht = lax.dot_general(w1_ref[...], x_ref[...], (((0,), (1,)), ((), ())),
                         preferred_element_type=_f32)
    htp = jnp.concatenate([ht, jnp.zeros((H, NPAD - N), _f32)], axis=1)
    gt_ref[...] = htp * dinv
    dinv_ref[...] = dinv


_mm1 = pl.pallas_call(
    _mm1_body,
    out_shape=[
        jax.ShapeDtypeStruct((16, NPAD), _f32),
        jax.ShapeDtypeStruct((16, NPAD), _f32),
    ],
)


def _comb1_body(s0_ref, s1_ref, gt_ref, dinv_ref, w2_ref, b1_ref, g2_ref):
    h1 = jnp.maximum(
        (s0_ref[...] + s1_ref[...] + gt_ref[...]) * dinv_ref[...]
        + b1_ref[...], 0.0)
    g2_ref[...] = lax.dot_general(
        w2_ref[...], h1, (((0,), (0,)), ((), ())),
        preferred_element_type=_f32) * dinv_ref[...]


_comb1 = pl.pallas_call(
    _comb1_body,
    out_shape=jax.ShapeDtypeStruct((16, NPAD), _f32),
)


def _comb2_body(q0_ref, q1_ref, g2_ref, dinv_ref, b2_ref, batch_ref,
                wl_ref, bl_ref, out_ref):
    h2 = jnp.maximum(
        (q0_ref[...] + q1_ref[...] + g2_ref[...]) * dinv_ref[...]
        + b2_ref[...], 0.0)
    iota = lax.broadcasted_iota(jnp.int32, (G, NPAD), 0)
    onehot = (iota == batch_ref[...]).astype(_f32)
    pooled = lax.dot_general(onehot, h2, (((1,), (1,)), ((), ())),
                             preferred_element_type=_f32)
    counts = lax.dot_general(onehot, jnp.ones((1, NPAD), _f32),
                             (((1,), (1,)), ((), ())),
                             preferred_element_type=_f32)
    pooled = pooled / jnp.maximum(counts, 1.0)
    z = jnp.dot(pooled, wl_ref[...], preferred_element_type=_f32) + bl_ref[...]
    out_ref[...] = jax.nn.sigmoid(z)


_comb2 = pl.pallas_call(
    _comb2_body,
    out_shape=jax.ShapeDtypeStruct((G, 1), _f32),
)


def kernel(x, edge_index, batch, W1, b1, W2, b2, Wl, bl):
    ei3 = edge_index.reshape(2, NCHUNKS, CHUNK)
    batr = jnp.pad(batch, (0, NPAD - N),
                   constant_values=G + 1).reshape(1, NPAD)
    zeros1 = jnp.zeros((NPAD,), _f32)
    zeros2 = jnp.zeros((NPAD, H), _f32)

    d0t, d1t = _deg(ei3, zeros1)
    g1t, dinvt = _mm1(x, W1, d0t, d1t)

    s0t, s1t, _ = _msg(ei3, g1t, zeros2)
    g2t = _comb1(s0t, s1t, g1t, dinvt, W2, b1.reshape(H, 1))

    q0t, q1t, _ = _msg(ei3, g2t, zeros2)
    out2d = _comb2(q0t, q1t, g2t, dinvt, b2.reshape(H, 1), batr,
                   Wl, bl.reshape(1, 1))
    return out2d[:, 0]


# speedup vs baseline: 1.8759x; 1.0920x over previous
import functools

import jax
import jax.numpy as jnp
from jax import lax
from jax.experimental import pallas as pl
from jax.experimental.pallas import tpu as pltpu
from jax.experimental.pallas import tpu_sc as plsc

N = 10000
E = 320000
D = 128
H = 16
G = 64
NPAD = 10240
CHUNK = 128
NCHUNKS = E // CHUNK
NW = 32
K78 = NCHUNKS // NW
TAIL = NCHUNKS - K78 * NW
NBUF = 4
RPS = NPAD // 16

_mesh = plsc.VectorSubcoreMesh(core_axis_name="c", subcore_axis_name="s")
_sc_params = pltpu.CompilerParams(use_tc_tiling_on_sc=False,
                                  needs_layout_passes=False)

_f32 = jnp.float32


@functools.partial(
    pl.kernel,
    out_type=[jax.ShapeDtypeStruct((16, NPAD), _f32),
              jax.ShapeDtypeStruct((16, NPAD), _f32)],
    mesh=_mesh,
    compiler_params=_sc_params,
    scratch_types=[
        pltpu.VMEM_SHARED((NPAD,), _f32),
        pltpu.VMEM((K78 + 1, CHUNK), jnp.int32),
        pltpu.VMEM((CHUNK,), _f32),
        pltpu.VMEM((RPS,), _f32),
        pltpu.VMEM((16, RPS), _f32),
        pltpu.SemaphoreType.DMA,
        pltpu.SemaphoreType.DMA,
        pltpu.SemaphoreType.DMA,
        pltpu.SemaphoreType.DMA,
    ],
)
def _deg(ei3_hbm, zeros1_hbm, degt0_hbm, degt1_hbm, deg_sh, didx, ones_v,
         dval_v, drep_v, sm0, sm1, sm2, sm3):
    ssem = (sm0, sm1, sm2, sm3)
    c = lax.axis_index("c")
    s = lax.axis_index("s")
    wid = s * 2 + c
    for k in range(CHUNK // 16):
        ones_v[pl.ds(k * 16, 16)] = jnp.full((16,), 1.0, _f32)
    pltpu.sync_copy(zeros1_hbm.at[pl.ds(s * RPS, RPS)],
                    deg_sh.at[pl.ds(s * RPS, RPS)])
    pltpu.sync_copy(ei3_hbm.at[1].at[pl.ds(wid * K78, K78)],
                    didx.at[pl.ds(0, K78)])

    @pl.when(wid < TAIL)
    def _():
        pltpu.sync_copy(ei3_hbm.at[1].at[K78 * NW + wid], didx.at[K78])

    kmax = jnp.where(wid < TAIL, K78 + 1, K78)
    plsc.subcore_barrier()

    def grp_body(gidx, carry):
        for b in range(NBUF):
            j = gidx * NBUF + b

            @pl.when(j < kmax)
            def _(b=b, j=j):
                @pl.when(j >= NBUF)
                def _():
                    pltpu.make_async_copy(
                        ones_v, deg_sh.at[didx.at[0]], ssem[b]).wait()
                pltpu.async_copy(ones_v, deg_sh.at[didx.at[j]], ssem[b],
                                 add=True)
        return carry

    lax.fori_loop(0, (K78 + 1 + NBUF - 1) // NBUF, grp_body, 0)
    for b in range(NBUF):
        pltpu.make_async_copy(ones_v, deg_sh.at[didx.at[0]], ssem[b]).wait()
    plsc.subcore_barrier()

    pltpu.sync_copy(deg_sh.at[pl.ds(s * RPS, RPS)], dval_v)

    @plsc.parallel_loop(0, RPS // 16, unroll=4)
    def rep_body(k):
        v = dval_v[pl.ds(k * 16, 16)]
        for l in range(16):
            drep_v[l, pl.ds(k * 16, 16)] = v

    @pl.when(c == 0)
    def _():
        pltpu.sync_copy(drep_v,
                        degt0_hbm.at[pl.ds(0, 16), pl.ds(s * RPS, RPS)])

    @pl.when(c == 1)
    def _():
        pltpu.sync_copy(drep_v,
                        degt1_hbm.at[pl.ds(0, 16), pl.ds(s * RPS, RPS)])


@functools.partial(
    pl.kernel,
    out_type=[jax.ShapeDtypeStruct((16, NPAD), _f32),
              jax.ShapeDtypeStruct((16, NPAD), _f32),
              jax.ShapeDtypeStruct((NPAD, H), _f32)],
    mesh=_mesh,
    compiler_params=_sc_params,
    scratch_types=[
        pltpu.VMEM_SHARED((NPAD, H), _f32),
        pltpu.VMEM((K78 + 1, CHUNK), jnp.int32),
        pltpu.VMEM((K78 + 1, CHUNK), jnp.int32),
        pltpu.VMEM((16, RPS), _f32),
        pltpu.VMEM((RPS, H), _f32),
        pltpu.VMEM((CHUNK, H), _f32),
        pltpu.VMEM((CHUNK, H), _f32),
        pltpu.VMEM((CHUNK, H), _f32),
        pltpu.VMEM((CHUNK, H), _f32),
        pltpu.SemaphoreType.DMA,
        pltpu.SemaphoreType.DMA,
        pltpu.SemaphoreType.DMA,
        pltpu.SemaphoreType.DMA,
        pltpu.SemaphoreType.DMA,
        pltpu.SemaphoreType.DMA,
        pltpu.SemaphoreType.DMA,
        pltpu.SemaphoreType.DMA,
    ],
)
def _msg(ei3_hbm, gt_hbm, zeros2_hbm, spt0_hbm, spt1_hbm, grows_hbm,
         acc_sh, sidx, didx, tbuf, rbuf, r0, r1, r2, r3,
         gm0, gm1, gm2, gm3, sm0, sm1, sm2, sm3):
    rows = (r0, r1, r2, r3)
    gsem = (gm0, gm1, gm2, gm3)
    ssem = (sm0, sm1, sm2, sm3)
    c = lax.axis_index("c")
    s = lax.axis_index("s")
    wid = s * 2 + c
    lane_iota = lax.iota(jnp.int32, 16)

    pltpu.sync_copy(gt_hbm.at[pl.ds(0, 16), pl.ds(s * RPS, RPS)], tbuf)

    @plsc.parallel_loop(0, RPS, unroll=16)
    def tr_body(j):
        col = lane_iota * 0 + j
        v = plsc.load_gather(tbuf, [lane_iota, col])
        rbuf[j, :] = v
    pltpu.sync_copy(rbuf, grows_hbm.at[pl.ds(s * RPS, RPS)])

    pltpu.sync_copy(zeros2_hbm.at[pl.ds(s * RPS, RPS)],
                    acc_sh.at[pl.ds(s * RPS, RPS)])
    pltpu.sync_copy(ei3_hbm.at[0].at[pl.ds(wid * K78, K78)],
                    sidx.at[pl.ds(0, K78)])
    pltpu.sync_copy(ei3_hbm.at[1].at[pl.ds(wid * K78, K78)],
                    didx.at[pl.ds(0, K78)])

    @pl.when(wid < TAIL)
    def _():
        pltpu.sync_copy(ei3_hbm.at[0].at[K78 * NW + wid], sidx.at[K78])
        pltpu.sync_copy(ei3_hbm.at[1].at[K78 * NW + wid], didx.at[K78])

    kmax = jnp.where(wid < TAIL, K78 + 1, K78)
    plsc.subcore_barrier()

    def issue_gather(j, b):
        pltpu.async_copy(grows_hbm.at[sidx.at[j]], rows[b], gsem[b])

    def wait_gather(b):
        pltpu.make_async_copy(grows_hbm.at[sidx.at[0]], rows[b],
                              gsem[b]).wait()

    def issue_scatter(j, b):
        pltpu.async_copy(rows[b], acc_sh.at[didx.at[j]], ssem[b], add=True)

    def wait_scatter(b):
        pltpu.make_async_copy(rows[b], acc_sh.at[didx.at[0]], ssem[b]).wait()

    for b in range(NBUF):
        issue_gather(jnp.int32(b), b)

    def grp_body(gidx, carry):
        for b in range(NBUF):
            j = gidx * NBUF + b

            @pl.when(j < kmax)
            def _(b=b, j=j):
                wait_gather(b)
                issue_scatter(j, b)
                jn = j + NBUF

                @pl.when(jn < kmax)
                def _(b=b, jn=jn):
                    wait_scatter(b)
                    issue_gather(jn, b)
        return carry

    lax.fori_loop(0, (K78 + 1 + NBUF - 1) // NBUF, grp_body, 0)
    for b in range(NBUF):
        wait_scatter(b)
    plsc.subcore_barrier()

    pltpu.sync_copy(acc_sh.at[pl.ds(s * RPS, RPS)], rbuf)

    @plsc.parallel_loop(0, RPS // 16, unroll=4)
    def trb_body(k):
        ridx = k * 16 + lane_iota
        for l in range(16):
            cidx = lane_iota * 0 + l
            v = plsc.load_gather(rbuf, [ridx, cidx])
            tbuf[l, pl.ds(k * 16, 16)] = v

    @pl.when(c == 0)
    def _():
        pltpu.sync_copy(tbuf,
                        spt0_hbm.at[pl.ds(0, 16), pl.ds(s * RPS, RPS)])

    @pl.when(c == 1)
    def _():
        pltpu.sync_copy(tbuf,
                        spt1_hbm.at[pl.ds(0, 16), pl.ds(s * RPS, RPS)])


def _mm1_body(x_ref, w1_ref, d0_ref, d1_ref, gt_ref, dinv_ref):
    deg = d0_ref[...] + d1_ref[...] + 1.0
    dinv = lax.rsqrt(jnp.maximum(deg, 1.0))
    ht = lax.dot_general(w1_ref[...], x_ref[...], (((0,), (1,)), ((), ())),
                         preferred_element_type=_f32)
    htp = jnp.concatenate([ht, jnp.zeros((H, NPAD - N), _f32)], axis=1)
    gt_ref[...] = htp * dinv
    dinv_ref[...] = dinv


_mm1 = pl.pallas_call(
    _mm1_body,
    out_shape=[
        jax.ShapeDtypeStruct((16, NPAD), _f32),
        jax.ShapeDtypeStruct((16, NPAD), _f32),
    ],
)


def _comb1_body(s0_ref, s1_ref, gt_ref, dinv_ref, w2_ref, b1_ref, g2_ref):
    h1 = jnp.maximum(
        (s0_ref[...] + s1_ref[...] + gt_ref[...]) * dinv_ref[...]
        + b1_ref[...], 0.0)
    g2_ref[...] = lax.dot_general(
        w2_ref[...], h1, (((0,), (0,)), ((), ())),
        preferred_element_type=_f32) * dinv_ref[...]


_comb1 = pl.pallas_call(
    _comb1_body,
    out_shape=jax.ShapeDtypeStruct((16, NPAD), _f32),
)


def _comb2_body(q0_ref, q1_ref, g2_ref, dinv_ref, b2_ref, batch_ref,
                wl_ref, bl_ref, out_ref):
    h2 = jnp.maximum(
        (q0_ref[...] + q1_ref[...] + g2_ref[...]) * dinv_ref[...]
        + b2_ref[...], 0.0)
    iota = lax.broadcasted_iota(jnp.int32, (G, NPAD), 0)
    onehot = (iota == batch_ref[...]).astype(_f32)
    pooled = lax.dot_general(onehot, h2, (((1,), (1,)), ((), ())),
                             preferred_element_type=_f32)
    counts = lax.dot_general(onehot, jnp.ones((1, NPAD), _f32),
                             (((1,), (1,)), ((), ())),
                             preferred_element_type=_f32)
    pooled = pooled / jnp.maximum(counts, 1.0)
    z = jnp.dot(pooled, wl_ref[...], preferred_element_type=_f32) + bl_ref[...]
    out_ref[...] = jax.nn.sigmoid(z)


_comb2 = pl.pallas_call(
    _comb2_body,
    out_shape=jax.ShapeDtypeStruct((G, 1), _f32),
)


def kernel(x, edge_index, batch, W1, b1, W2, b2, Wl, bl):
    ei3 = edge_index.reshape(2, NCHUNKS, CHUNK)
    batr = jnp.pad(batch, (0, NPAD - N),
                   constant_values=G + 1).reshape(1, NPAD)
    zeros1 = jnp.zeros((NPAD,), _f32)
    zeros2 = jnp.zeros((NPAD, H), _f32)

    d0t, d1t = _deg(ei3, zeros1)
    g1t, dinvt = _mm1(x, W1, d0t, d1t)

    s0t, s1t, _ = _msg(ei3, g1t, zeros2)
    g2t = _comb1(s0t, s1t, g1t, dinvt, W2, b1.reshape(H, 1))

    q0t, q1t, _ = _msg(ei3, g2t, zeros2)
    out2d = _comb2(q0t, q1t, g2t, dinvt, b2.reshape(H, 1), batr,
                   Wl, bl.reshape(1, 1))
    return out2d[:, 0]


# speedup vs baseline: 1.9202x; 1.0236x over previous
import functools

import jax
import jax.numpy as jnp
from jax import lax
from jax.experimental import pallas as pl
from jax.experimental.pallas import tpu as pltpu
from jax.experimental.pallas import tpu_sc as plsc

N = 10000
E = 320000
D = 128
H = 16
G = 64
NPAD = 10240
CHUNK = 128
NCHUNKS = E // CHUNK
NW = 32
K78 = NCHUNKS // NW
TAIL = NCHUNKS - K78 * NW
NBUF = 4
RPS = NPAD // 16

_mesh = plsc.VectorSubcoreMesh(core_axis_name="c", subcore_axis_name="s")
_sc_params = pltpu.CompilerParams(use_tc_tiling_on_sc=False,
                                  needs_layout_passes=False)

_f32 = jnp.float32


@functools.partial(
    pl.kernel,
    out_type=[jax.ShapeDtypeStruct((16, NPAD), _f32),
              jax.ShapeDtypeStruct((16, NPAD), _f32)],
    mesh=_mesh,
    compiler_params=_sc_params,
    scratch_types=[
        pltpu.VMEM_SHARED((NPAD,), _f32),
        pltpu.VMEM((K78 + 1, CHUNK), jnp.int32),
        pltpu.VMEM((CHUNK,), _f32),
        pltpu.VMEM((RPS,), _f32),
        pltpu.VMEM((16, RPS), _f32),
        pltpu.SemaphoreType.DMA,
        pltpu.SemaphoreType.DMA,
        pltpu.SemaphoreType.DMA,
        pltpu.SemaphoreType.DMA,
    ],
)
def _deg(ei3_hbm, degt0_hbm, degt1_hbm, deg_sh, didx, ones_v,
         dval_v, drep_v, sm0, sm1, sm2, sm3):
    ssem = (sm0, sm1, sm2, sm3)
    c = lax.axis_index("c")
    s = lax.axis_index("s")
    wid = s * 2 + c
    for k in range(CHUNK // 16):
        ones_v[pl.ds(k * 16, 16)] = jnp.full((16,), 1.0, _f32)

    @plsc.parallel_loop(0, RPS // 16, unroll=4)
    def zero_body(k):
        dval_v[pl.ds(k * 16, 16)] = jnp.zeros((16,), _f32)

    pltpu.sync_copy(dval_v, deg_sh.at[pl.ds(s * RPS, RPS)])
    pltpu.sync_copy(ei3_hbm.at[1].at[pl.ds(wid * K78, K78)],
                    didx.at[pl.ds(0, K78)])

    @pl.when(wid < TAIL)
    def _():
        pltpu.sync_copy(ei3_hbm.at[1].at[K78 * NW + wid], didx.at[K78])

    kmax = jnp.where(wid < TAIL, K78 + 1, K78)
    plsc.subcore_barrier()

    def grp_body(gidx, carry):
        for b in range(NBUF):
            j = gidx * NBUF + b

            @pl.when(j < kmax)
            def _(b=b, j=j):
                @pl.when(j >= NBUF)
                def _():
                    pltpu.make_async_copy(
                        ones_v, deg_sh.at[didx.at[0]], ssem[b]).wait()
                pltpu.async_copy(ones_v, deg_sh.at[didx.at[j]], ssem[b],
                                 add=True)
        return carry

    lax.fori_loop(0, (K78 + 1 + NBUF - 1) // NBUF, grp_body, 0)
    for b in range(NBUF):
        pltpu.make_async_copy(ones_v, deg_sh.at[didx.at[0]], ssem[b]).wait()
    plsc.subcore_barrier()

    pltpu.sync_copy(deg_sh.at[pl.ds(s * RPS, RPS)], dval_v)

    @plsc.parallel_loop(0, RPS // 16, unroll=4)
    def rep_body(k):
        v = dval_v[pl.ds(k * 16, 16)]
        for l in range(16):
            drep_v[l, pl.ds(k * 16, 16)] = v

    @pl.when(c == 0)
    def _():
        pltpu.sync_copy(drep_v,
                        degt0_hbm.at[pl.ds(0, 16), pl.ds(s * RPS, RPS)])

    @pl.when(c == 1)
    def _():
        pltpu.sync_copy(drep_v,
                        degt1_hbm.at[pl.ds(0, 16), pl.ds(s * RPS, RPS)])


@functools.partial(
    pl.kernel,
    out_type=[jax.ShapeDtypeStruct((16, NPAD), _f32),
              jax.ShapeDtypeStruct((16, NPAD), _f32),
              jax.ShapeDtypeStruct((NPAD, H), _f32)],
    mesh=_mesh,
    compiler_params=_sc_params,
    scratch_types=[
        pltpu.VMEM_SHARED((NPAD, H), _f32),
        pltpu.VMEM((K78 + 1, CHUNK), jnp.int32),
        pltpu.VMEM((K78 + 1, CHUNK), jnp.int32),
        pltpu.VMEM((16, RPS), _f32),
        pltpu.VMEM((RPS, H), _f32),
        pltpu.VMEM((CHUNK, H), _f32),
        pltpu.VMEM((CHUNK, H), _f32),
        pltpu.VMEM((CHUNK, H), _f32),
        pltpu.VMEM((CHUNK, H), _f32),
        pltpu.SemaphoreType.DMA,
        pltpu.SemaphoreType.DMA,
        pltpu.SemaphoreType.DMA,
        pltpu.SemaphoreType.DMA,
        pltpu.SemaphoreType.DMA,
        pltpu.SemaphoreType.DMA,
        pltpu.SemaphoreType.DMA,
        pltpu.SemaphoreType.DMA,
    ],
)
def _msg(ei3_hbm, gt_hbm, spt0_hbm, spt1_hbm, grows_hbm,
         acc_sh, sidx, didx, tbuf, rbuf, r0, r1, r2, r3,
         gm0, gm1, gm2, gm3, sm0, sm1, sm2, sm3):
    rows = (r0, r1, r2, r3)
    gsem = (gm0, gm1, gm2, gm3)
    ssem = (sm0, sm1, sm2, sm3)
    c = lax.axis_index("c")
    s = lax.axis_index("s")
    wid = s * 2 + c
    lane_iota = lax.iota(jnp.int32, 16)

    pltpu.sync_copy(gt_hbm.at[pl.ds(0, 16), pl.ds(s * RPS, RPS)], tbuf)

    @plsc.parallel_loop(0, RPS, unroll=16)
    def tr_body(j):
        col = lane_iota * 0 + j
        v = plsc.load_gather(tbuf, [lane_iota, col])
        rbuf[j, :] = v
    pltpu.sync_copy(rbuf, grows_hbm.at[pl.ds(s * RPS, RPS)])

    @plsc.parallel_loop(0, RPS, unroll=16)
    def zero_body(j):
        rbuf[j, :] = jnp.zeros((16,), _f32)

    pltpu.sync_copy(rbuf, acc_sh.at[pl.ds(s * RPS, RPS)])
    pltpu.sync_copy(ei3_hbm.at[0].at[pl.ds(wid * K78, K78)],
                    sidx.at[pl.ds(0, K78)])
    pltpu.sync_copy(ei3_hbm.at[1].at[pl.ds(wid * K78, K78)],
                    didx.at[pl.ds(0, K78)])

    @pl.when(wid < TAIL)
    def _():
        pltpu.sync_copy(ei3_hbm.at[0].at[K78 * NW + wid], sidx.at[K78])
        pltpu.sync_copy(ei3_hbm.at[1].at[K78 * NW + wid], didx.at[K78])

    kmax = jnp.where(wid < TAIL, K78 + 1, K78)
    plsc.subcore_barrier()

    def issue_gather(j, b):
        pltpu.async_copy(grows_hbm.at[sidx.at[j]], rows[b], gsem[b])

    def wait_gather(b):
        pltpu.make_async_copy(grows_hbm.at[sidx.at[0]], rows[b],
                              gsem[b]).wait()

    def issue_scatter(j, b):
        pltpu.async_copy(rows[b], acc_sh.at[didx.at[j]], ssem[b], add=True)

    def wait_scatter(b):
        pltpu.make_async_copy(rows[b], acc_sh.at[didx.at[0]], ssem[b]).wait()

    for b in range(NBUF):
        issue_gather(jnp.int32(b), b)

    def grp_body(gidx, carry):
        for b in range(NBUF):
            j = gidx * NBUF + b

            @pl.when(j < kmax)
            def _(b=b, j=j):
                wait_gather(b)
                issue_scatter(j, b)
                jn = j + NBUF

                @pl.when(jn < kmax)
                def _(b=b, jn=jn):
                    wait_scatter(b)
                    issue_gather(jn, b)
        return carry

    lax.fori_loop(0, (K78 + 1 + NBUF - 1) // NBUF, grp_body, 0)
    for b in range(NBUF):
        wait_scatter(b)
    plsc.subcore_barrier()

    pltpu.sync_copy(acc_sh.at[pl.ds(s * RPS, RPS)], rbuf)

    @plsc.parallel_loop(0, RPS // 16, unroll=4)
    def trb_body(k):
        ridx = k * 16 + lane_iota
        for l in range(16):
            cidx = lane_iota * 0 + l
            v = plsc.load_gather(rbuf, [ridx, cidx])
            tbuf[l, pl.ds(k * 16, 16)] = v

    @pl.when(c == 0)
    def _():
        pltpu.sync_copy(tbuf,
                        spt0_hbm.at[pl.ds(0, 16), pl.ds(s * RPS, RPS)])

    @pl.when(c == 1)
    def _():
        pltpu.sync_copy(tbuf,
                        spt1_hbm.at[pl.ds(0, 16), pl.ds(s * RPS, RPS)])


def _mm1_body(x_ref, w1_ref, d0_ref, d1_ref, gt_ref, dinv_ref):
    deg = d0_ref[...] + d1_ref[...] + 1.0
    dinv = lax.rsqrt(jnp.maximum(deg, 1.0))
    ht = lax.dot_general(w1_ref[...], x_ref[...], (((0,), (1,)), ((), ())),
                         preferred_element_type=_f32)
    htp = jnp.concatenate([ht, jnp.zeros((H, NPAD - N), _f32)], axis=1)
    gt_ref[...] = htp * dinv
    dinv_ref[...] = dinv


_mm1 = pl.pallas_call(
    _mm1_body,
    out_shape=[
        jax.ShapeDtypeStruct((16, NPAD), _f32),
        jax.ShapeDtypeStruct((16, NPAD), _f32),
    ],
)


def _comb1_body(s0_ref, s1_ref, gt_ref, dinv_ref, w2_ref, b1_ref, g2_ref):
    h1 = jnp.maximum(
        (s0_ref[...] + s1_ref[...] + gt_ref[...]) * dinv_ref[...]
        + b1_ref[...], 0.0)
    g2_ref[...] = lax.dot_general(
        w2_ref[...], h1, (((0,), (0,)), ((), ())),
        preferred_element_type=_f32) * dinv_ref[...]


_comb1 = pl.pallas_call(
    _comb1_body,
    out_shape=jax.ShapeDtypeStruct((16, NPAD), _f32),
)


def _comb2_body(q0_ref, q1_ref, g2_ref, dinv_ref, b2_ref, batch_ref,
                wl_ref, bl_ref, out_ref):
    h2 = jnp.maximum(
        (q0_ref[...] + q1_ref[...] + g2_ref[...]) * dinv_ref[...]
        + b2_ref[...], 0.0)
    iota = lax.broadcasted_iota(jnp.int32, (G, NPAD), 0)
    onehot = (iota == batch_ref[...]).astype(_f32)
    pooled = lax.dot_general(onehot, h2, (((1,), (1,)), ((), ())),
                             preferred_element_type=_f32)
    counts = lax.dot_general(onehot, jnp.ones((1, NPAD), _f32),
                             (((1,), (1,)), ((), ())),
                             preferred_element_type=_f32)
    pooled = pooled / jnp.maximum(counts, 1.0)
    z = jnp.dot(pooled, wl_ref[...], preferred_element_type=_f32) + bl_ref[...]
    out_ref[...] = jax.nn.sigmoid(z)


_comb2 = pl.pallas_call(
    _comb2_body,
    out_shape=jax.ShapeDtypeStruct((G, 1), _f32),
)


def kernel(x, edge_index, batch, W1, b1, W2, b2, Wl, bl):
    ei3 = edge_index.reshape(2, NCHUNKS, CHUNK)
    batr = jnp.pad(batch, (0, NPAD - N),
                   constant_values=G + 1).reshape(1, NPAD)

    d0t, d1t = _deg(ei3)
    g1t, dinvt = _mm1(x, W1, d0t, d1t)

    s0t, s1t, _ = _msg(ei3, g1t)
    g2t = _comb1(s0t, s1t, g1t, dinvt, W2, b1.reshape(H, 1))

    q0t, q1t, _ = _msg(ei3, g2t)
    out2d = _comb2(q0t, q1t, g2t, dinvt, b2.reshape(H, 1), batr,
                   Wl, bl.reshape(1, 1))
    return out2d[:, 0]
